# baseline (speedup 1.0000x reference)
import functools

import jax
import jax.numpy as jnp
from jax import lax
from jax.experimental import pallas as pl
from jax.experimental.pallas import tpu as pltpu
from jax.experimental.pallas import tpu_sc as plsc

BATCH = 4096
SEQ = 200
HIDDEN = 64
VOCAB = 1000000
NC = 2
NS = 16
NW = NC * NS
BPW = BATCH // NW
NSTEPS = SEQ // 2

B1 = 0.40047579449240756
B2 = -0.06803906174166623
B3 = 0.010041876871370552
B4 = -0.0009390641370112668
B5 = 3.8726031922469735e-05


def _gelu_vec(x):
    x2 = x * x
    p = B1 + x2 * (B2 + x2 * (B3 + x2 * (B4 + x2 * B5)))
    u = jnp.minimum(jnp.maximum(0.5 + x * p, 0.0), 1.0)
    return x * u


def _body(seq_hbm, wt_hbm, pos_hbm, t5_hbm,
          seq_v, rows_a, rows_b, out_a, out_b, pos_v,
          gs_a, gs_b, ws_a, ws_b):
    wid = lax.axis_index("s") * NC + lax.axis_index("c")
    pltpu.sync_copy(pos_hbm, pos_v)
    pltpu.sync_copy(seq_hbm.at[wid], seq_v)

    def issue_gather(r, rows):
        pltpu.async_copy(wt_hbm.at[seq_v.at[r]], rows[0], rows[1])

    def wait_gather(rows):
        pltpu.make_async_copy(wt_hbm.at[seq_v.at[0]], rows[0], rows[1]).wait()

    def issue_wb(r, out):
        pltpu.async_copy(out[0], t5_hbm.at[r, :, wid], out[1])

    def wait_wb(out):
        pltpu.make_async_copy(out[0], t5_hbm.at[0, :, 0], out[1]).wait()

    def compute(r, rows, out):
        rows_v = rows[0]
        out_v = out[0]
        iota = lax.iota(jnp.int32, 16)
        pbase = r * HIDDEN

        def body_c(c0i, carry):
            c0 = 16 * c0i
            for j in range(16):
                perm = (iota + j) & 15
                colv = perm + c0
                posv = plsc.load_gather(pos_v, [colv + pbase])
                iv = colv >> 3
                rv = colv & 7
                for k0 in range(8):
                    kvec = iota + 16 * k0
                    val = plsc.load_gather(rows_v, [kvec, colv])
                    plsc.store_scatter(out_v, [iv, rv, kvec],
                                       _gelu_vec(val + posv))
            return carry

        lax.fori_loop(0, HIDDEN // 16, body_c, 0)

    A = (rows_a, gs_a)
    B = (rows_b, gs_b)
    OA = (out_a, ws_a)
    OB = (out_b, ws_b)

    issue_gather(0, A)

    def step(s, carry):
        r0 = 2 * s

        @pl.when(s > 0)
        def _():
            wait_wb(OB)

        issue_gather(r0 + 1, B)
        with jax.named_scope("gwaitA"):
            wait_gather(A)
        with jax.named_scope("cmpA"):
            compute(r0, A, OA)
        issue_wb(r0, OA)
        with jax.named_scope("gwaitB"):
            wait_gather(B)
        with jax.named_scope("cmpB"):
            compute(r0 + 1, B, OB)
        issue_wb(r0 + 1, OB)
        with jax.named_scope("wwaitA"):
            wait_wb(OA)

        @pl.when(s < NSTEPS - 1)
        def _():
            issue_gather(r0 + 2, A)

        return carry

    lax.fori_loop(0, NSTEPS, step, 0)
    wait_wb(OB)


def kernel(input_seq, word_table, pos_table):
    wt128 = jnp.pad(word_table, ((0, 0), (0, 128 - HIDDEN)))
    seq_t = input_seq.astype(jnp.int32).reshape(NW, BPW, SEQ).transpose(0, 2, 1)
    pos_f = pos_table.reshape(SEQ * HIDDEN)
    mesh = plsc.VectorSubcoreMesh(core_axis_name="c", subcore_axis_name="s")
    run = functools.partial(
        pl.kernel,
        mesh=mesh,
        out_type=jax.ShapeDtypeStruct((SEQ, 8, NW, 8, BPW), jnp.float32),
        compiler_params=pltpu.CompilerParams(
            use_tc_tiling_on_sc=False, needs_layout_passes=False),
        scratch_types=[
            pltpu.VMEM((SEQ, BPW), jnp.int32),
            pltpu.VMEM((BPW, 128), jnp.float32),
            pltpu.VMEM((BPW, 128), jnp.float32),
            pltpu.VMEM((8, 8, BPW), jnp.float32),
            pltpu.VMEM((8, 8, BPW), jnp.float32),
            pltpu.VMEM((SEQ * HIDDEN,), jnp.float32),
            pltpu.SemaphoreType.DMA,
            pltpu.SemaphoreType.DMA,
            pltpu.SemaphoreType.DMA,
            pltpu.SemaphoreType.DMA,
        ],
    )(_body)
    t5 = run(seq_t, wt128, pos_f)
    return jnp.transpose(t5, (2, 4, 0, 1, 3)).reshape(BATCH, SEQ, HIDDEN)


# speedup vs baseline: 1.5485x; 1.5485x over previous
import functools

import jax
import jax.numpy as jnp
from jax import lax
from jax.experimental import pallas as pl
from jax.experimental.pallas import tpu as pltpu
from jax.experimental.pallas import tpu_sc as plsc

BATCH = 4096
SEQ = 200
HIDDEN = 64
VOCAB = 1000000
NC = 2
NS = 16
NW = NC * NS
BPW = BATCH // NW
NCHUNK = BPW * SEQ // 128
NSTEPS = NCHUNK // 2

B1 = 0.40047579449240756
B2 = -0.06803906174166623
B3 = 0.010041876871370552
B4 = -0.0009390641370112668
B5 = 3.8726031922469735e-05


def _gelu_vec(x):
    x2 = x * x
    p = B1 + x2 * (B2 + x2 * (B3 + x2 * (B4 + x2 * B5)))
    u = jnp.minimum(jnp.maximum(0.5 + x * p, 0.0), 1.0)
    return x * u


def _body(seq_hbm, wt_hbm, pos_hbm, out_hbm,
          seq_v, rows_a, rows_b, out_a, out_b, pos_v,
          gs_a, gs_b, ws_a, ws_b):
    wid = lax.axis_index("s") * NC + lax.axis_index("c")
    pltpu.sync_copy(pos_hbm, pos_v)
    pltpu.sync_copy(seq_hbm.at[wid], seq_v)

    def issue_gather(c, rows):
        pltpu.async_copy(wt_hbm.at[seq_v.at[c]], rows[0], rows[1])

    def wait_gather(rows):
        pltpu.make_async_copy(wt_hbm.at[seq_v.at[0]], rows[0], rows[1]).wait()

    def issue_wb(c, out):
        pltpu.async_copy(out[0], out_hbm.at[wid, c], out[1])

    def wait_wb(out):
        pltpu.make_async_copy(out[0], out_hbm.at[0, 0], out[1]).wait()

    def compute(c, rows, out):
        rows_v = rows[0]
        out_v = out[0]
        p0 = lax.rem(c * 128, SEQ)

        def body_g(g, carry):
            for kk in range(8):
                k = 8 * g + kk
                praw = p0 + k
                p = jnp.where(praw >= SEQ, praw - SEQ, praw)
                pb = p * HIDDEN
                for cc in range(HIDDEN // 16):
                    sl = pl.ds(16 * cc, 16)
                    x = rows_v[k, sl] + pos_v[pl.ds(pb + 16 * cc, 16)]
                    out_v[4 * g + kk // 2,
                          pl.ds(64 * (kk % 2) + 16 * cc, 16)] = _gelu_vec(x)
            return carry

        lax.fori_loop(0, 16, body_g, 0)

    A = (rows_a, gs_a)
    B = (rows_b, gs_b)
    OA = (out_a, ws_a)
    OB = (out_b, ws_b)

    issue_gather(0, A)

    def step(s, carry):
        c0 = 2 * s

        @pl.when(s > 0)
        def _():
            wait_wb(OB)

        issue_gather(c0 + 1, B)
        with jax.named_scope("gwaitA"):
            wait_gather(A)
        with jax.named_scope("cmpA"):
            compute(c0, A, OA)
        issue_wb(c0, OA)
        with jax.named_scope("gwaitB"):
            wait_gather(B)
        with jax.named_scope("cmpB"):
            compute(c0 + 1, B, OB)
        issue_wb(c0 + 1, OB)
        with jax.named_scope("wwaitA"):
            wait_wb(OA)

        @pl.when(s < NSTEPS - 1)
        def _():
            issue_gather(c0 + 2, A)

        return carry

    lax.fori_loop(0, NSTEPS, step, 0)
    wait_wb(OB)


def kernel(input_seq, word_table, pos_table):
    wt128 = jnp.pad(word_table, ((0, 0), (0, 128 - HIDDEN)))
    seq_c = input_seq.astype(jnp.int32).reshape(NW, NCHUNK, 128)
    pos_f = pos_table.reshape(SEQ * HIDDEN)
    mesh = plsc.VectorSubcoreMesh(core_axis_name="c", subcore_axis_name="s")
    run = functools.partial(
        pl.kernel,
        mesh=mesh,
        out_type=jax.ShapeDtypeStruct((NW, NCHUNK, 64, 128), jnp.float32),
        compiler_params=pltpu.CompilerParams(needs_layout_passes=False),
        scratch_types=[
            pltpu.VMEM((NCHUNK, 128), jnp.int32),
            pltpu.VMEM((128, 128), jnp.float32),
            pltpu.VMEM((128, 128), jnp.float32),
            pltpu.VMEM((64, 128), jnp.float32),
            pltpu.VMEM((64, 128), jnp.float32),
            pltpu.VMEM((SEQ * HIDDEN,), jnp.float32),
            pltpu.SemaphoreType.DMA,
            pltpu.SemaphoreType.DMA,
            pltpu.SemaphoreType.DMA,
            pltpu.SemaphoreType.DMA,
        ],
    )(_body)
    out4 = run(seq_c, wt128, pos_f)
    return out4.reshape(BATCH, SEQ, HIDDEN)


# speedup vs baseline: 3.3107x; 2.1381x over previous
import functools

import jax
import jax.numpy as jnp
from jax import lax
from jax.experimental import pallas as pl
from jax.experimental.pallas import tpu as pltpu
from jax.experimental.pallas import tpu_sc as plsc

BATCH = 4096
SEQ = 200
HIDDEN = 64
NC = 2
NS = 16
NW = NC * NS
SEQ_PER_W = BATCH // NW
NSTEPS = SEQ_PER_W // 2
HALF = SEQ // 2

B1 = 0.40047579449240756
B2 = -0.06803906174166623
B3 = 0.010041876871370552
B4 = -0.0009390641370112668
B5 = 3.8726031922469735e-05


def _gelu_vec(x):
    x2 = x * x
    p = B1 + x2 * (B2 + x2 * (B3 + x2 * (B4 + x2 * B5)))
    u = jnp.minimum(jnp.maximum(0.5 + x * p, 0.0), 1.0)
    return x * u


def _body(seq_hbm, wt_hbm, pt_hbm, out_hbm, idx_v, rows_v, pos_v,
          gs0, gs1, ws0, ws1):
    wid = lax.axis_index("s") * NC + lax.axis_index("c")
    pltpu.sync_copy(pt_hbm, pos_v)
    pltpu.sync_copy(seq_hbm.at[wid], idx_v)

    def issue_gather(g, buf, sem):
        for j in range(2):
            pltpu.async_copy(
                wt_hbm.at[idx_v.at[g, j]],
                rows_v.at[buf, pl.ds(j * HALF, HALF)],
                sem)

    def wait_gather(buf, sem):
        for j in range(2):
            pltpu.make_async_copy(
                wt_hbm.at[idx_v.at[0, j]],
                rows_v.at[buf, pl.ds(j * HALF, HALF)],
                sem).wait()

    def issue_wb(g, buf, sem):
        pltpu.async_copy(rows_v.at[buf], out_hbm.at[wid * SEQ_PER_W + g], sem)

    def wait_wb(buf, sem):
        pltpu.make_async_copy(rows_v.at[buf], out_hbm.at[0], sem).wait()

    def compute(buf):
        def body(i, c):
            for rr in range(4):
                r = i * 4 + rr
                for cc in range(HIDDEN // 16):
                    sl = pl.ds(cc * 16, 16)
                    x = rows_v[buf, r, sl] + pos_v[r, sl]
                    rows_v[buf, r, sl] = _gelu_vec(x)
            return c
        lax.fori_loop(0, SEQ // 4, body, 0)

    issue_gather(0, 0, gs0)

    def step(s, carry):
        g0 = 2 * s

        @pl.when(s > 0)
        def _():
            wait_wb(1, ws1)

        issue_gather(g0 + 1, 1, gs1)
        wait_gather(0, gs0)
        compute(0)
        issue_wb(g0, 0, ws0)
        wait_gather(1, gs1)
        compute(1)
        issue_wb(g0 + 1, 1, ws1)
        wait_wb(0, ws0)

        @pl.when(s < NSTEPS - 1)
        def _():
            issue_gather(g0 + 2, 0, gs0)

        return carry

    lax.fori_loop(0, NSTEPS, step, 0)
    wait_wb(1, ws1)


def kernel(input_seq, word_table, pos_table):
    seq4 = input_seq.astype(jnp.int32).reshape(NW, SEQ_PER_W, 2, HALF)
    mesh = plsc.VectorSubcoreMesh(core_axis_name="c", subcore_axis_name="s")
    run = functools.partial(
        pl.kernel,
        mesh=mesh,
        out_type=jax.ShapeDtypeStruct((BATCH, SEQ, HIDDEN), jnp.float32),
        compiler_params=pltpu.CompilerParams(use_tc_tiling_on_sc=False),
        scratch_types=[
            pltpu.VMEM((SEQ_PER_W, 2, HALF), jnp.int32),
            pltpu.VMEM((2, SEQ, HIDDEN), jnp.float32),
            pltpu.VMEM((SEQ, HIDDEN), jnp.float32),
            pltpu.SemaphoreType.DMA,
            pltpu.SemaphoreType.DMA,
            pltpu.SemaphoreType.DMA,
            pltpu.SemaphoreType.DMA,
        ],
    )(_body)
    return run(seq4, word_table, pos_table)


# speedup vs baseline: 3.4341x; 1.0372x over previous
import functools

import jax
import jax.numpy as jnp
from jax import lax
from jax.experimental import pallas as pl
from jax.experimental.pallas import tpu as pltpu
from jax.experimental.pallas import tpu_sc as plsc

BATCH = 4096
SEQ = 200
HIDDEN = 64
NC = 2
NS = 16
NW = NC * NS
SEQ_PER_W = BATCH // NW
NSTEPS = SEQ_PER_W // 2
HALF = SEQ // 2

C1 = -1.5957691216057308
C2 = C1 * 0.044715


def _gelu_vec(x):
    return x / (1.0 + jnp.exp(x * (C1 + C2 * (x * x))))


def _body(seq_hbm, wt_hbm, pt_hbm, out_hbm, idx_v, rows_v, pos_v,
          gs0, gs1, ws0, ws1):
    wid = lax.axis_index("s") * NC + lax.axis_index("c")
    pltpu.sync_copy(pt_hbm, pos_v)
    pltpu.sync_copy(seq_hbm.at[wid], idx_v)

    def issue_gather(g, buf, sem):
        for j in range(2):
            pltpu.async_copy(
                wt_hbm.at[idx_v.at[g, j]],
                rows_v.at[buf, pl.ds(j * HALF, HALF)],
                sem)

    def wait_gather(buf, sem):
        for j in range(2):
            pltpu.make_async_copy(
                wt_hbm.at[idx_v.at[0, j]],
                rows_v.at[buf, pl.ds(j * HALF, HALF)],
                sem).wait()

    def issue_wb(g, buf, sem):
        pltpu.async_copy(rows_v.at[buf], out_hbm.at[wid * SEQ_PER_W + g], sem)

    def wait_wb(buf, sem):
        pltpu.make_async_copy(rows_v.at[buf], out_hbm.at[0], sem).wait()

    def compute(buf):
        def body(i, c):
            for rr in range(4):
                r = i * 4 + rr
                for cc in range(HIDDEN // 16):
                    sl = pl.ds(cc * 16, 16)
                    x = rows_v[buf, r, sl] + pos_v[r, sl]
                    rows_v[buf, r, sl] = _gelu_vec(x)
            return c
        lax.fori_loop(0, SEQ // 4, body, 0)

    issue_gather(0, 0, gs0)

    def step(s, carry):
        g0 = 2 * s

        @pl.when(s > 0)
        def _():
            wait_wb(1, ws1)

        issue_gather(g0 + 1, 1, gs1)
        wait_gather(0, gs0)
        compute(0)
        issue_wb(g0, 0, ws0)
        wait_gather(1, gs1)
        compute(1)
        issue_wb(g0 + 1, 1, ws1)
        wait_wb(0, ws0)

        @pl.when(s < NSTEPS - 1)
        def _():
            issue_gather(g0 + 2, 0, gs0)

        return carry

    lax.fori_loop(0, NSTEPS, step, 0)
    wait_wb(1, ws1)


def kernel(input_seq, word_table, pos_table):
    seq4 = input_seq.astype(jnp.int32).reshape(NW, SEQ_PER_W, 2, HALF)
    mesh = plsc.VectorSubcoreMesh(core_axis_name="c", subcore_axis_name="s")
    run = functools.partial(
        pl.kernel,
        mesh=mesh,
        out_type=jax.ShapeDtypeStruct((BATCH, SEQ, HIDDEN), jnp.float32),
        compiler_params=pltpu.CompilerParams(use_tc_tiling_on_sc=False),
        scratch_types=[
            pltpu.VMEM((SEQ_PER_W, 2, HALF), jnp.int32),
            pltpu.VMEM((2, SEQ, HIDDEN), jnp.float32),
            pltpu.VMEM((SEQ, HIDDEN), jnp.float32),
            pltpu.SemaphoreType.DMA,
            pltpu.SemaphoreType.DMA,
            pltpu.SemaphoreType.DMA,
            pltpu.SemaphoreType.DMA,
        ],
    )(_body)
    return run(seq4, word_table, pos_table)


# speedup vs baseline: 3.5849x; 1.0439x over previous
import functools

import jax
import jax.numpy as jnp
from jax import lax
from jax.experimental import pallas as pl
from jax.experimental.pallas import tpu as pltpu
from jax.experimental.pallas import tpu_sc as plsc

BATCH = 4096
SEQ = 200
HIDDEN = 64
NC = 2
NS = 16
NW = NC * NS
SEQ_PER_W = BATCH // NW
NITERS = SEQ_PER_W // 4
HALF = SEQ // 2

C1 = -1.5957691216057308
C2 = C1 * 0.044715


def _gelu_vec(x):
    return x / (1.0 + jnp.exp(x * (C1 + C2 * (x * x))))


def _body(seq_hbm, wt_hbm, pt_hbm, out_hbm, idx_v, rows_v, pos_v,
          gs0, gs1, gs2, gs3, ws0, ws1, ws2, ws3):
    wid = lax.axis_index("s") * NC + lax.axis_index("c")
    pltpu.sync_copy(pt_hbm, pos_v)
    pltpu.sync_copy(seq_hbm.at[wid], idx_v)

    gsems = (gs0, gs1, gs2, gs3)
    wsems = (ws0, ws1, ws2, ws3)

    def issue_gather(g, buf):
        for j in range(2):
            pltpu.async_copy(
                wt_hbm.at[idx_v.at[g, j]],
                rows_v.at[buf, pl.ds(j * HALF, HALF)],
                gsems[buf])

    def wait_gather(buf):
        for j in range(2):
            pltpu.make_async_copy(
                wt_hbm.at[idx_v.at[0, j]],
                rows_v.at[buf, pl.ds(j * HALF, HALF)],
                gsems[buf]).wait()

    def issue_wb(g, buf):
        pltpu.async_copy(rows_v.at[buf], out_hbm.at[wid * SEQ_PER_W + g],
                         wsems[buf])

    def wait_wb(buf):
        pltpu.make_async_copy(rows_v.at[buf], out_hbm.at[0], wsems[buf]).wait()

    def compute(buf):
        def body(i, c):
            for rr in range(4):
                r = i * 4 + rr
                for cc in range(HIDDEN // 16):
                    sl = pl.ds(cc * 16, 16)
                    x = rows_v[buf, r, sl] + pos_v[r, sl]
                    rows_v[buf, r, sl] = _gelu_vec(x)
            return c
        lax.fori_loop(0, SEQ // 4, body, 0)

    issue_gather(0, 0)
    issue_gather(1, 1)

    def step(t, carry):
        g0 = 4 * t

        @pl.when(t > 0)
        def _():
            wait_wb(2)
        issue_gather(g0 + 2, 2)

        @pl.when(t > 0)
        def _():
            wait_wb(3)
        issue_gather(g0 + 3, 3)

        wait_gather(0)
        compute(0)
        issue_wb(g0, 0)

        wait_gather(1)
        compute(1)
        issue_wb(g0 + 1, 1)

        wait_gather(2)
        compute(2)
        issue_wb(g0 + 2, 2)

        @pl.when(t < NITERS - 1)
        def _():
            wait_wb(0)
            issue_gather(g0 + 4, 0)

        wait_gather(3)
        compute(3)
        issue_wb(g0 + 3, 3)

        @pl.when(t < NITERS - 1)
        def _():
            wait_wb(1)
            issue_gather(g0 + 5, 1)

        return carry

    lax.fori_loop(0, NITERS, step, 0)
    wait_wb(0)
    wait_wb(1)
    wait_wb(2)
    wait_wb(3)


def kernel(input_seq, word_table, pos_table):
    seq4 = input_seq.astype(jnp.int32).reshape(NW, SEQ_PER_W, 2, HALF)
    mesh = plsc.VectorSubcoreMesh(core_axis_name="c", subcore_axis_name="s")
    run = functools.partial(
        pl.kernel,
        mesh=mesh,
        out_type=jax.ShapeDtypeStruct((BATCH, SEQ, HIDDEN), jnp.float32),
        compiler_params=pltpu.CompilerParams(use_tc_tiling_on_sc=False),
        scratch_types=[
            pltpu.VMEM((SEQ_PER_W, 2, HALF), jnp.int32),
            pltpu.VMEM((4, SEQ, HIDDEN), jnp.float32),
            pltpu.VMEM((SEQ, HIDDEN), jnp.float32),
            pltpu.SemaphoreType.DMA,
            pltpu.SemaphoreType.DMA,
            pltpu.SemaphoreType.DMA,
            pltpu.SemaphoreType.DMA,
            pltpu.SemaphoreType.DMA,
            pltpu.SemaphoreType.DMA,
            pltpu.SemaphoreType.DMA,
            pltpu.SemaphoreType.DMA,
        ],
    )(_body)
    return run(seq4, word_table, pos_table)
